# baseline (speedup 1.0000x reference)
import functools

import jax
import jax.numpy as jnp
from jax import lax
from jax.experimental import pallas as pl
from jax.experimental.pallas import tpu as pltpu
from jax.experimental.pallas import tpu_sc as plsc

N = 10000
E = 320000
F = 128
G = 64
C = 4
NB = 400
NBLK = N // NB
NSC = 2
NTEC = 16
ECH = 128
NCH = 160
EPAD = NTEC * NCH * ECH
NPAD = 10240


def _sc_mesh():
    return plsc.VectorSubcoreMesh(core_axis_name="c", subcore_axis_name="s",
                                  num_cores=NSC, num_subcores=NTEC)


def _writeout_split(copy_fn):
    sid = lax.axis_index("s")

    @pl.when(sid < NTEC - 1)
    def _():
        copy_fn(sid * 640, 640)

    @pl.when(sid == NTEC - 1)
    def _():
        copy_fn((NTEC - 1) * 640, N - (NTEC - 1) * 640)


def _root_kernel(first, x, roots, idxv, rows, sem):
    cid = lax.axis_index("c")
    sid = lax.axis_index("s")

    @pl.when(jnp.logical_and(cid == 0, sid == 0))
    def _():
        pltpu.sync_copy(first, idxv)
        pltpu.async_copy(x.at[idxv], rows, sem).wait()
        pltpu.sync_copy(rows, roots)


def _sc_root_gather(first, x):
    k = pl.kernel(
        _root_kernel,
        out_type=jax.ShapeDtypeStruct((G, F), jnp.float32),
        mesh=_sc_mesh(),
        scratch_types=[
            pltpu.VMEM((G,), jnp.int32),
            pltpu.VMEM((G, F), jnp.float32),
            pltpu.SemaphoreType.DMA,
        ],
    )
    return k(first, x)


def _agg_kernel(p, s0, d0, s1, d1, zrows,
                out, acc, rows_a, rows_b,
                sidx_0, sidx_1, sidx_2, sidx_3,
                didx_0, didx_1, didx_2, didx_3,
                gsem_a, gsem_b, ssem_a, ssem_b,
                isem_0, isem_1, isem_2, isem_3):
    cid = lax.axis_index("c")
    sid = lax.axis_index("s")
    pltpu.sync_copy(zrows, acc.at[pl.ds(sid * 640, 640)])
    plsc.subcore_barrier()

    for d in (0, 1):
        s_in = s0 if d == 0 else s1
        d_in = d0 if d == 0 else d1

        @pl.when(cid == d)
        def _():
            pd = p.at[d]
            rows = (rows_a, rows_b)
            gsem = (gsem_a, gsem_b)
            ssem = (ssem_a, ssem_b)
            sidx = (sidx_0, sidx_1, sidx_2, sidx_3)
            didx = (didx_0, didx_1, didx_2, didx_3)
            isem = (isem_0, isem_1, isem_2, isem_3)

            def fire_idx(j, ib):
                off = (sid * NCH + j) * ECH
                pltpu.async_copy(s_in.at[pl.ds(off, ECH)], sidx[ib], isem[ib])
                pltpu.async_copy(d_in.at[pl.ds(off, ECH)], didx[ib], isem[ib])

            def wait_idx(ib):
                pltpu.make_async_copy(s_in.at[pl.ds(0, ECH)], sidx[ib],
                                      isem[ib]).wait()
                pltpu.make_async_copy(d_in.at[pl.ds(0, ECH)], didx[ib],
                                      isem[ib]).wait()

            def fire_gather(rb, ib):
                pltpu.async_copy(pd.at[sidx[ib]], rows[rb], gsem[rb])

            def wait_gather(rb, ib):
                pltpu.make_async_copy(pd.at[sidx[ib]], rows[rb],
                                      gsem[rb]).wait()

            def scatter(rb, ib):
                pltpu.sync_copy(rows[rb], acc.at[didx[ib]], add=True)

            fire_idx(0, 0)
            fire_idx(1, 1)
            wait_idx(0)
            fire_gather(0, 0)
            fire_idx(2, 2)

            @pl.loop(0, (NCH - 4) // 4)
            def _(j):
                for k in range(4):
                    c = 4 * j + k + 1
                    rb = 1 - k % 2
                    ib = (k + 1) % 4
                    wait_idx(ib)
                    fire_gather(rb, ib)
                    fire_idx(c + 2, (k + 3) % 4)
                    wait_gather(1 - rb, k % 4)
                    scatter(1 - rb, k % 4)

            wait_idx(1)
            fire_gather(1, 1)
            fire_idx(159, 3)
            wait_gather(0, 0)
            scatter(0, 0)
            wait_idx(2)
            fire_gather(0, 2)
            wait_gather(1, 1)
            scatter(1, 1)
            wait_idx(3)
            fire_gather(1, 3)
            wait_gather(0, 2)
            scatter(0, 2)
            wait_gather(1, 3)
            scatter(1, 3)

    plsc.subcore_barrier()

    for d in (0, 1):
        @pl.when(cid == d)
        def _():
            _writeout_split(
                lambda r0, n: pltpu.sync_copy(acc.at[pl.ds(r0, n)],
                                              out.at[d].at[pl.ds(r0, n)]))


def _sc_edge_agg(p, s0, d0, s1, d1, zrows):
    k = pl.kernel(
        _agg_kernel,
        out_type=jax.ShapeDtypeStruct((2, N, F), jnp.float32),
        mesh=_sc_mesh(),
        scratch_types=(
            [pltpu.VMEM_SHARED((NPAD, F), jnp.float32)]
            + [pltpu.VMEM((ECH, F), jnp.float32)] * 2
            + [pltpu.VMEM((ECH,), jnp.int32)] * 8
            + [pltpu.SemaphoreType.DMA] * 8
        ),
    )
    return k(p, s0, d0, s1, d1, zrows)


EB = 6400
EBLK = E // EB
DD = 100


def _hist_kernel(ei, dh):
    i = pl.program_id(0)

    @pl.when(i == 0)
    def _():
        dh[...] = jnp.zeros_like(dh)

    ia = lax.broadcasted_iota(jnp.int32, (DD, EB), 0).astype(jnp.bfloat16)
    for d in (0, 1):
        dst = ei[1 - d:2 - d, :]
        a = dst // DD
        b = dst - a * DD
        abf = a.astype(jnp.bfloat16)
        bbf = b.astype(jnp.bfloat16)
        oha = (abf == ia).astype(jnp.bfloat16)
        ohb = (bbf == ia).astype(jnp.bfloat16)
        dh[d] += lax.dot_general(ohb, oha, (((1,), (1,)), ((), ())),
                                 preferred_element_type=jnp.float32)


def _tc_deghist(ei):
    return pl.pallas_call(
        _hist_kernel,
        grid=(EBLK,),
        in_specs=[pl.BlockSpec((2, EB), lambda i: (0, i))],
        out_specs=pl.BlockSpec((2, DD, DD), lambda i: (0, 0, 0)),
        out_shape=jax.ShapeDtypeStruct((2, DD, DD), jnp.float32),
    )(ei)


def _prep_kernel(ei, s0, d0, s1, d1):
    e0 = ei[0:1, :]
    e1 = ei[1:2, :]
    it = lax.broadcasted_iota(jnp.int32, (1, EPAD - E), 1)
    padz = it % 128
    padn = N + it % (NPAD - N)
    s0[...] = jnp.concatenate([e0, padz], axis=1).reshape(EPAD)
    d0[...] = jnp.concatenate([e1, padn], axis=1).reshape(EPAD)
    s1[...] = jnp.concatenate([e1, padz], axis=1).reshape(EPAD)
    d1[...] = jnp.concatenate([e0, padn], axis=1).reshape(EPAD)


def _tc_edgeprep(ei):
    return pl.pallas_call(
        _prep_kernel,
        out_shape=[jax.ShapeDtypeStruct((EPAD,), jnp.int32)] * 4,
    )(ei)


def _extract_deg(dmat, i):
    lane = lax.broadcasted_iota(jnp.int32, (DD, DD), 1)
    cols = []
    for k in range(NB // DD):
        sel = (lane == (NB // DD) * i + k).astype(jnp.float32)
        cols.append(jnp.sum(dmat * sel, axis=1, keepdims=True))
    return jnp.concatenate(cols, axis=0)


def _hscale_kernel(x, wcat, dh, p, dis):
    i = pl.program_id(0)
    hb = jnp.dot(x[...], wcat[...], preferred_element_type=jnp.float32)
    for d in (0, 1):
        deg = _extract_deg(dh[d], i)
        dv = jax.lax.rsqrt(deg + 1.0)
        p[d] = hb[:, d * F:(d + 1) * F] * dv
        dis[d] = dv


def _tc_hscale(x, wcat, dh):
    return pl.pallas_call(
        _hscale_kernel,
        grid=(NBLK,),
        in_specs=[
            pl.BlockSpec((NB, F), lambda i: (i, 0)),
            pl.BlockSpec((F, 2 * F), lambda i: (0, 0)),
            pl.BlockSpec((2, DD, DD), lambda i: (0, 0, 0)),
        ],
        out_specs=[
            pl.BlockSpec((2, NB, F), lambda i: (0, i, 0)),
            pl.BlockSpec((2, NB, 1), lambda i: (0, i, 0)),
        ],
        out_shape=[
            jax.ShapeDtypeStruct((2, N, F), jnp.float32),
            jax.ShapeDtypeStruct((2, N, 1), jnp.float32),
        ],
    )(x, wcat, dh)


def _first_kernel(bc, first, cnt, cacc):
    i = pl.program_id(0)

    @pl.when(i == 0)
    def _():
        cacc[...] = jnp.zeros_like(cacc)

    gidx = lax.broadcasted_iota(jnp.int32, (NB, G), 1)
    oh = (bc[...] == gidx).astype(jnp.float32)
    cacc[...] += jnp.sum(oh, axis=0, keepdims=True)

    @pl.when(i == NBLK - 1)
    def _():
        c = cacc[...]
        gj = lax.broadcasted_iota(jnp.int32, (G, G), 0)
        gg = lax.broadcasted_iota(jnp.int32, (G, G), 1)
        lt = (gj < gg).astype(jnp.float32)
        f = jnp.dot(c, lt, preferred_element_type=jnp.float32)
        first[...] = jnp.clip(f.astype(jnp.int32), 0, N - 1)
        cnt[...] = c


def _tc_first(batch_c):
    return pl.pallas_call(
        _first_kernel,
        grid=(NBLK,),
        in_specs=[pl.BlockSpec((NB, 1), lambda i: (i, 0))],
        out_specs=[
            pl.BlockSpec((1, G), lambda i: (0, 0)),
            pl.BlockSpec((1, G), lambda i: (0, 0)),
        ],
        out_shape=[
            jax.ShapeDtypeStruct((1, G), jnp.int32),
            jax.ShapeDtypeStruct((1, G), jnp.float32),
        ],
        scratch_shapes=[pltpu.VMEM((1, G), jnp.float32)],
    )(batch_c)


def _q_kernel(roots, wbot, q):
    q[...] = jnp.dot(jax.nn.relu(roots[...]), wbot[...],
                     preferred_element_type=jnp.float32)


def _tc_q(roots, wbot_cat):
    return pl.pallas_call(
        _q_kernel,
        out_shape=jax.ShapeDtypeStruct((G, 2 * F), jnp.float32),
    )(roots, wbot_cat)


def _layer2_kernel(pp, agg, dis, bc, q, wtop, b1, b2, p2, basev):
    b = bc[...]
    gidx = lax.broadcasted_iota(jnp.int32, (NB, G), 1)
    oh = (b == gidx).astype(jnp.float32)
    for d in (0, 1):
        dd = dis[d]
        conv1 = dd * (agg[d] + pp[d]) + b1[d]
        t = jax.nn.relu(conv1)
        h2 = (jnp.dot(t, wtop[d], preferred_element_type=jnp.float32)
              + jnp.dot(oh, q[:, d * F:(d + 1) * F],
                        preferred_element_type=jnp.float32))
        p2d = dd * h2
        p2[d] = p2d
        basev[d] = dd * p2d + b2[d]


def _tc_layer2(p, agg, dis, batch_c, q, wtop, b1, b2):
    return pl.pallas_call(
        _layer2_kernel,
        grid=(NBLK,),
        in_specs=[
            pl.BlockSpec((2, NB, F), lambda i: (0, i, 0)),
            pl.BlockSpec((2, NB, F), lambda i: (0, i, 0)),
            pl.BlockSpec((2, NB, 1), lambda i: (0, i, 0)),
            pl.BlockSpec((NB, 1), lambda i: (i, 0)),
            pl.BlockSpec((G, 2 * F), lambda i: (0, 0)),
            pl.BlockSpec((2, F, F), lambda i: (0, 0, 0)),
            pl.BlockSpec((2, F), lambda i: (0, 0)),
            pl.BlockSpec((2, F), lambda i: (0, 0)),
        ],
        out_specs=[
            pl.BlockSpec((2, NB, F), lambda i: (0, i, 0)),
            pl.BlockSpec((2, NB, F), lambda i: (0, i, 0)),
        ],
        out_shape=[
            jax.ShapeDtypeStruct((2, N, F), jnp.float32),
            jax.ShapeDtypeStruct((2, N, F), jnp.float32),
        ],
    )(p, agg, dis, batch_c, q, wtop, b1, b2)


def _final_kernel(agg2, basev, dis, bc, cnt, lw, lb, out, acc):
    i = pl.program_id(0)

    @pl.when(i == 0)
    def _():
        acc[...] = jnp.zeros_like(acc)

    b = bc[...]
    gidx = lax.broadcasted_iota(jnp.int32, (NB, G), 1)
    oh = (b == gidx).astype(jnp.float32)
    for d in (0, 1):
        h = jax.nn.relu(dis[d] * agg2[d] + basev[d])
        contrib = lax.dot_general(oh, h, (((0,), (0,)), ((), ())),
                                  preferred_element_type=jnp.float32)
        acc[:, d * F:(d + 1) * F] += contrib

    @pl.when(i == NBLK - 1)
    def _():
        gr = lax.broadcasted_iota(jnp.int32, (G, G), 0)
        gc = lax.broadcasted_iota(jnp.int32, (G, G), 1)
        dm = (gr == gc).astype(jnp.float32) / jnp.maximum(cnt[...], 1.0)
        mean = jnp.dot(dm, acc[...], preferred_element_type=jnp.float32)
        logits = jnp.dot(mean, lw[...],
                         preferred_element_type=jnp.float32) + lb[...]
        m = jnp.max(logits, axis=-1, keepdims=True)
        z = logits - m
        out[...] = z - jnp.log(jnp.sum(jnp.exp(z), axis=-1, keepdims=True))


def _tc_final(agg2, basev, dis, batch_c, cnt, lw, lb):
    return pl.pallas_call(
        _final_kernel,
        grid=(NBLK,),
        in_specs=[
            pl.BlockSpec((2, NB, F), lambda i: (0, i, 0)),
            pl.BlockSpec((2, NB, F), lambda i: (0, i, 0)),
            pl.BlockSpec((2, NB, 1), lambda i: (0, i, 0)),
            pl.BlockSpec((NB, 1), lambda i: (i, 0)),
            pl.BlockSpec((1, G), lambda i: (0, 0)),
            pl.BlockSpec((2 * F, C), lambda i: (0, 0)),
            pl.BlockSpec((1, C), lambda i: (0, 0)),
        ],
        out_specs=pl.BlockSpec((G, C), lambda i: (0, 0)),
        out_shape=jax.ShapeDtypeStruct((G, C), jnp.float32),
        scratch_shapes=[pltpu.VMEM((G, 2 * F), jnp.float32)],
    )(agg2, basev, dis, batch_c, cnt, lw, lb)


def kernel(x, edge_index, batch, bu1_W, bu1_b, td1_W, td1_b, root_W, root_b,
           bu2_W, bu2_b, td2_W, td2_b, lin_W, lin_b):
    ei = edge_index.astype(jnp.int32)
    batch_c = batch.astype(jnp.int32).reshape(N, 1)
    zrows = jnp.zeros((640, F), jnp.float32)

    wcat = jnp.concatenate([bu1_W, td1_W], axis=1)
    wtop = jnp.stack([bu2_W[:F], td2_W[:F]])
    wbot_cat = jnp.concatenate([bu2_W[F:], td2_W[F:]], axis=1)
    b1 = jnp.stack([bu1_b, td1_b])
    b2 = jnp.stack([bu2_b, td2_b])

    dh = _tc_deghist(ei)
    p, dis = _tc_hscale(x, wcat, dh)
    first, cnt = _tc_first(batch_c)

    roots = _sc_root_gather(first.reshape(G), x)
    q = _tc_q(roots, wbot_cat)

    s0, d0, s1, d1 = _tc_edgeprep(ei)

    agg = _sc_edge_agg(p, s0, d0, s1, d1, zrows)
    p2, basev = _tc_layer2(p, agg, dis, batch_c, q, wtop, b1, b2)
    agg2 = _sc_edge_agg(p2, s0, d0, s1, d1, zrows)
    return _tc_final(agg2, basev, dis, batch_c, cnt,
                     lin_W, lin_b.reshape(1, C))


# speedup vs baseline: 1.0522x; 1.0522x over previous
import functools

import jax
import jax.numpy as jnp
from jax import lax
from jax.experimental import pallas as pl
from jax.experimental.pallas import tpu as pltpu
from jax.experimental.pallas import tpu_sc as plsc

N = 10000
E = 320000
F = 128
G = 64
C = 4
NB = 400
NBLK = N // NB
NSC = 2
NTEC = 16
ECH = 128
NCH = 160
EPAD = NTEC * NCH * ECH
NPAD = 10240


def _sc_mesh():
    return plsc.VectorSubcoreMesh(core_axis_name="c", subcore_axis_name="s",
                                  num_cores=NSC, num_subcores=NTEC)


def _writeout_split(copy_fn):
    sid = lax.axis_index("s")

    @pl.when(sid < NTEC - 1)
    def _():
        copy_fn(sid * 640, 640)

    @pl.when(sid == NTEC - 1)
    def _():
        copy_fn((NTEC - 1) * 640, N - (NTEC - 1) * 640)


def _root_kernel(first, x, roots, idxv, rows, sem):
    cid = lax.axis_index("c")
    sid = lax.axis_index("s")

    @pl.when(jnp.logical_and(cid == 0, sid == 0))
    def _():
        pltpu.sync_copy(first, idxv)
        pltpu.async_copy(x.at[idxv], rows, sem).wait()
        pltpu.sync_copy(rows, roots)


def _sc_root_gather(first, x):
    k = pl.kernel(
        _root_kernel,
        out_type=jax.ShapeDtypeStruct((G, F), jnp.float32),
        mesh=_sc_mesh(),
        scratch_types=[
            pltpu.VMEM((G,), jnp.int32),
            pltpu.VMEM((G, F), jnp.float32),
            pltpu.SemaphoreType.DMA,
        ],
    )
    return k(first, x)


def _agg_kernel(p, e0, e1, sp, dp, zrows,
                out, acc, rows_a, rows_b,
                sidx_0, sidx_1, sidx_2, sidx_3,
                didx_0, didx_1, didx_2, didx_3,
                gsem_a, gsem_b, ssem_a, ssem_b,
                isem_0, isem_1, isem_2, isem_3):
    cid = lax.axis_index("c")
    sid = lax.axis_index("s")
    pltpu.sync_copy(zrows, acc.at[pl.ds(sid * 640, 640)])
    plsc.subcore_barrier()

    for d in (0, 1):
        s_in = e0 if d == 0 else e1
        d_in = e1 if d == 0 else e0

        @pl.when(cid == d)
        def _():
            pd = p.at[d]
            rows = (rows_a, rows_b)
            gsem = (gsem_a, gsem_b)
            ssem = (ssem_a, ssem_b)
            sidx = (sidx_0, sidx_1, sidx_2, sidx_3)
            didx = (didx_0, didx_1, didx_2, didx_3)
            isem = (isem_0, isem_1, isem_2, isem_3)

            def fire_idx(j, ib):
                off = (sid * NCH + j) * ECH

                @pl.when(off < E)
                def _():
                    pltpu.async_copy(s_in.at[pl.ds(off, ECH)],
                                     sidx[ib], isem[ib])
                    pltpu.async_copy(d_in.at[pl.ds(off, ECH)],
                                     didx[ib], isem[ib])

                @pl.when(off >= E)
                def _():
                    poff = off - E
                    pltpu.async_copy(sp.at[pl.ds(poff, ECH)],
                                     sidx[ib], isem[ib])
                    pltpu.async_copy(dp.at[pl.ds(poff, ECH)],
                                     didx[ib], isem[ib])

            def wait_idx(ib):
                pltpu.make_async_copy(s_in.at[pl.ds(0, ECH)], sidx[ib],
                                      isem[ib]).wait()
                pltpu.make_async_copy(d_in.at[pl.ds(0, ECH)], didx[ib],
                                      isem[ib]).wait()

            def fire_gather(rb, ib):
                pltpu.async_copy(pd.at[sidx[ib]], rows[rb], gsem[rb])

            def wait_gather(rb, ib):
                pltpu.make_async_copy(pd.at[sidx[ib]], rows[rb],
                                      gsem[rb]).wait()

            def scatter(rb, ib):
                pltpu.sync_copy(rows[rb], acc.at[didx[ib]], add=True)

            fire_idx(0, 0)
            fire_idx(1, 1)
            wait_idx(0)
            fire_gather(0, 0)
            fire_idx(2, 2)

            @pl.loop(0, (NCH - 4) // 4)
            def _(j):
                for k in range(4):
                    c = 4 * j + k + 1
                    rb = 1 - k % 2
                    ib = (k + 1) % 4
                    wait_idx(ib)
                    fire_gather(rb, ib)
                    fire_idx(c + 2, (k + 3) % 4)
                    wait_gather(1 - rb, k % 4)
                    scatter(1 - rb, k % 4)

            wait_idx(1)
            fire_gather(1, 1)
            fire_idx(159, 3)
            wait_gather(0, 0)
            scatter(0, 0)
            wait_idx(2)
            fire_gather(0, 2)
            wait_gather(1, 1)
            scatter(1, 1)
            wait_idx(3)
            fire_gather(1, 3)
            wait_gather(0, 2)
            scatter(0, 2)
            wait_gather(1, 3)
            scatter(1, 3)

    plsc.subcore_barrier()

    for d in (0, 1):
        @pl.when(cid == d)
        def _():
            _writeout_split(
                lambda r0, n: pltpu.sync_copy(acc.at[pl.ds(r0, n)],
                                              out.at[d].at[pl.ds(r0, n)]))


def _sc_edge_agg(p, e0, e1, sp, dp, zrows):
    k = pl.kernel(
        _agg_kernel,
        out_type=jax.ShapeDtypeStruct((2, N, F), jnp.float32),
        mesh=_sc_mesh(),
        scratch_types=(
            [pltpu.VMEM_SHARED((NPAD, F), jnp.float32)]
            + [pltpu.VMEM((ECH, F), jnp.float32)] * 2
            + [pltpu.VMEM((ECH,), jnp.int32)] * 8
            + [pltpu.SemaphoreType.DMA] * 8
        ),
    )
    return k(p, e0, e1, sp, dp, zrows)


EB = 3200
EBLK = E // EB
DD = 100


def _hist_kernel(ei, dh):
    i = pl.program_id(0)

    @pl.when(i == 0)
    def _():
        dh[...] = jnp.zeros_like(dh)

    ia = lax.broadcasted_iota(jnp.int32, (DD, EB), 0)
    for d in (0, 1):
        dst = ei[1 - d:2 - d, :]
        a = dst // DD
        b = dst - a * DD
        oha = (a == ia).astype(jnp.bfloat16)
        ohb = (b == ia).astype(jnp.bfloat16)
        dh[d] += lax.dot_general(ohb, oha, (((1,), (1,)), ((), ())),
                                 preferred_element_type=jnp.float32)


def _tc_deghist(ei):
    return pl.pallas_call(
        _hist_kernel,
        grid=(EBLK,),
        in_specs=[pl.BlockSpec((2, EB), lambda i: (0, i))],
        out_specs=pl.BlockSpec((2, DD, DD), lambda i: (0, 0, 0)),
        out_shape=jax.ShapeDtypeStruct((2, DD, DD), jnp.float32),
    )(ei)


def _prep_kernel(ei, e0lin, e1lin):
    e0lin[...] = ei[0:1, :].reshape(E)
    e1lin[...] = ei[1:2, :].reshape(E)


def _tc_edgeprep(ei):
    return pl.pallas_call(
        _prep_kernel,
        out_shape=[jax.ShapeDtypeStruct((E,), jnp.int32)] * 2,
    )(ei)


def _extract_deg(dmat, i):
    lane = lax.broadcasted_iota(jnp.int32, (DD, DD), 1)
    cols = []
    for k in range(NB // DD):
        sel = (lane == (NB // DD) * i + k).astype(jnp.float32)
        cols.append(jnp.sum(dmat * sel, axis=1, keepdims=True))
    return jnp.concatenate(cols, axis=0)


def _hscale_kernel(x, wcat, dh, p, dis):
    i = pl.program_id(0)
    hb = jnp.dot(x[...], wcat[...], preferred_element_type=jnp.float32)
    for d in (0, 1):
        deg = _extract_deg(dh[d], i)
        dv = jax.lax.rsqrt(deg + 1.0)
        p[d] = hb[:, d * F:(d + 1) * F] * dv
        dis[d] = dv


def _tc_hscale(x, wcat, dh):
    return pl.pallas_call(
        _hscale_kernel,
        grid=(NBLK,),
        in_specs=[
            pl.BlockSpec((NB, F), lambda i: (i, 0)),
            pl.BlockSpec((F, 2 * F), lambda i: (0, 0)),
            pl.BlockSpec((2, DD, DD), lambda i: (0, 0, 0)),
        ],
        out_specs=[
            pl.BlockSpec((2, NB, F), lambda i: (0, i, 0)),
            pl.BlockSpec((2, NB, 1), lambda i: (0, i, 0)),
        ],
        out_shape=[
            jax.ShapeDtypeStruct((2, N, F), jnp.float32),
            jax.ShapeDtypeStruct((2, N, 1), jnp.float32),
        ],
    )(x, wcat, dh)


def _first_kernel(bc, first, cnt, cacc):
    i = pl.program_id(0)

    @pl.when(i == 0)
    def _():
        cacc[...] = jnp.zeros_like(cacc)

    gidx = lax.broadcasted_iota(jnp.int32, (NB, G), 1)
    oh = (bc[...] == gidx).astype(jnp.float32)
    cacc[...] += jnp.sum(oh, axis=0, keepdims=True)

    @pl.when(i == NBLK - 1)
    def _():
        c = cacc[...]
        gj = lax.broadcasted_iota(jnp.int32, (G, G), 0)
        gg = lax.broadcasted_iota(jnp.int32, (G, G), 1)
        lt = (gj < gg).astype(jnp.float32)
        f = jnp.dot(c, lt, preferred_element_type=jnp.float32)
        first[...] = jnp.clip(f.astype(jnp.int32), 0, N - 1)
        cnt[...] = c


def _tc_first(batch_c):
    return pl.pallas_call(
        _first_kernel,
        grid=(NBLK,),
        in_specs=[pl.BlockSpec((NB, 1), lambda i: (i, 0))],
        out_specs=[
            pl.BlockSpec((1, G), lambda i: (0, 0)),
            pl.BlockSpec((1, G), lambda i: (0, 0)),
        ],
        out_shape=[
            jax.ShapeDtypeStruct((1, G), jnp.int32),
            jax.ShapeDtypeStruct((1, G), jnp.float32),
        ],
        scratch_shapes=[pltpu.VMEM((1, G), jnp.float32)],
    )(batch_c)


def _q_kernel(roots, wbot, q):
    q[...] = jnp.dot(jax.nn.relu(roots[...]), wbot[...],
                     preferred_element_type=jnp.float32)


def _tc_q(roots, wbot_cat):
    return pl.pallas_call(
        _q_kernel,
        out_shape=jax.ShapeDtypeStruct((G, 2 * F), jnp.float32),
    )(roots, wbot_cat)


def _layer2_kernel(pp, agg, dis, bc, q, wtop, b1, b2, p2, basev):
    b = bc[...]
    gidx = lax.broadcasted_iota(jnp.int32, (NB, G), 1)
    oh = (b == gidx).astype(jnp.float32)
    for d in (0, 1):
        dd = dis[d]
        conv1 = dd * (agg[d] + pp[d]) + b1[d]
        t = jax.nn.relu(conv1)
        h2 = (jnp.dot(t, wtop[d], preferred_element_type=jnp.float32)
              + jnp.dot(oh, q[:, d * F:(d + 1) * F],
                        preferred_element_type=jnp.float32))
        p2d = dd * h2
        p2[d] = p2d
        basev[d] = dd * p2d + b2[d]


def _tc_layer2(p, agg, dis, batch_c, q, wtop, b1, b2):
    return pl.pallas_call(
        _layer2_kernel,
        grid=(NBLK,),
        in_specs=[
            pl.BlockSpec((2, NB, F), lambda i: (0, i, 0)),
            pl.BlockSpec((2, NB, F), lambda i: (0, i, 0)),
            pl.BlockSpec((2, NB, 1), lambda i: (0, i, 0)),
            pl.BlockSpec((NB, 1), lambda i: (i, 0)),
            pl.BlockSpec((G, 2 * F), lambda i: (0, 0)),
            pl.BlockSpec((2, F, F), lambda i: (0, 0, 0)),
            pl.BlockSpec((2, F), lambda i: (0, 0)),
            pl.BlockSpec((2, F), lambda i: (0, 0)),
        ],
        out_specs=[
            pl.BlockSpec((2, NB, F), lambda i: (0, i, 0)),
            pl.BlockSpec((2, NB, F), lambda i: (0, i, 0)),
        ],
        out_shape=[
            jax.ShapeDtypeStruct((2, N, F), jnp.float32),
            jax.ShapeDtypeStruct((2, N, F), jnp.float32),
        ],
    )(p, agg, dis, batch_c, q, wtop, b1, b2)


def _final_kernel(agg2, basev, dis, bc, cnt, lw, lb, out, acc):
    i = pl.program_id(0)

    @pl.when(i == 0)
    def _():
        acc[...] = jnp.zeros_like(acc)

    b = bc[...]
    gidx = lax.broadcasted_iota(jnp.int32, (NB, G), 1)
    oh = (b == gidx).astype(jnp.float32)
    for d in (0, 1):
        h = jax.nn.relu(dis[d] * agg2[d] + basev[d])
        contrib = lax.dot_general(oh, h, (((0,), (0,)), ((), ())),
                                  preferred_element_type=jnp.float32)
        acc[:, d * F:(d + 1) * F] += contrib

    @pl.when(i == NBLK - 1)
    def _():
        gr = lax.broadcasted_iota(jnp.int32, (G, G), 0)
        gc = lax.broadcasted_iota(jnp.int32, (G, G), 1)
        dm = (gr == gc).astype(jnp.float32) / jnp.maximum(cnt[...], 1.0)
        mean = jnp.dot(dm, acc[...], preferred_element_type=jnp.float32)
        logits = jnp.dot(mean, lw[...],
                         preferred_element_type=jnp.float32) + lb[...]
        m = jnp.max(logits, axis=-1, keepdims=True)
        z = logits - m
        out[...] = z - jnp.log(jnp.sum(jnp.exp(z), axis=-1, keepdims=True))


def _tc_final(agg2, basev, dis, batch_c, cnt, lw, lb):
    return pl.pallas_call(
        _final_kernel,
        grid=(NBLK,),
        in_specs=[
            pl.BlockSpec((2, NB, F), lambda i: (0, i, 0)),
            pl.BlockSpec((2, NB, F), lambda i: (0, i, 0)),
            pl.BlockSpec((2, NB, 1), lambda i: (0, i, 0)),
            pl.BlockSpec((NB, 1), lambda i: (i, 0)),
            pl.BlockSpec((1, G), lambda i: (0, 0)),
            pl.BlockSpec((2 * F, C), lambda i: (0, 0)),
            pl.BlockSpec((1, C), lambda i: (0, 0)),
        ],
        out_specs=pl.BlockSpec((G, C), lambda i: (0, 0)),
        out_shape=jax.ShapeDtypeStruct((G, C), jnp.float32),
        scratch_shapes=[pltpu.VMEM((G, 2 * F), jnp.float32)],
    )(agg2, basev, dis, batch_c, cnt, lw, lb)


def kernel(x, edge_index, batch, bu1_W, bu1_b, td1_W, td1_b, root_W, root_b,
           bu2_W, bu2_b, td2_W, td2_b, lin_W, lin_b):
    ei = edge_index.astype(jnp.int32)
    batch_c = batch.astype(jnp.int32).reshape(N, 1)
    zrows = jnp.zeros((640, F), jnp.float32)

    wcat = jnp.concatenate([bu1_W, td1_W], axis=1)
    wtop = jnp.stack([bu2_W[:F], td2_W[:F]])
    wbot_cat = jnp.concatenate([bu2_W[F:], td2_W[F:]], axis=1)
    b1 = jnp.stack([bu1_b, td1_b])
    b2 = jnp.stack([bu2_b, td2_b])

    dh = _tc_deghist(ei)
    e0lin, e1lin = _tc_edgeprep(ei)
    p, dis = _tc_hscale(x, wcat, dh)
    first, cnt = _tc_first(batch_c)

    roots = _sc_root_gather(first.reshape(G), x)
    q = _tc_q(roots, wbot_cat)

    it = jnp.arange(EPAD - E, dtype=jnp.int32)
    sp = it % 128
    dp = N + it % (NPAD - N)

    agg = _sc_edge_agg(p, e0lin, e1lin, sp, dp, zrows)
    p2, basev = _tc_layer2(p, agg, dis, batch_c, q, wtop, b1, b2)
    agg2 = _sc_edge_agg(p2, e0lin, e1lin, sp, dp, zrows)
    return _tc_final(agg2, basev, dis, batch_c, cnt,
                     lin_W, lin_b.reshape(1, C))


# speedup vs baseline: 1.0661x; 1.0132x over previous
import functools

import jax
import jax.numpy as jnp
from jax import lax
from jax.experimental import pallas as pl
from jax.experimental.pallas import tpu as pltpu
from jax.experimental.pallas import tpu_sc as plsc

N = 10000
E = 320000
F = 128
G = 64
C = 4
NB = 400
NBLK = N // NB
NSC = 2
NTEC = 16
ECH = 128
NCH = 160
EPAD = NTEC * NCH * ECH
NPAD = 10240


def _sc_mesh():
    return plsc.VectorSubcoreMesh(core_axis_name="c", subcore_axis_name="s",
                                  num_cores=NSC, num_subcores=NTEC)


def _writeout_split(copy_fn):
    sid = lax.axis_index("s")

    @pl.when(sid < NTEC - 1)
    def _():
        copy_fn(sid * 640, 640)

    @pl.when(sid == NTEC - 1)
    def _():
        copy_fn((NTEC - 1) * 640, N - (NTEC - 1) * 640)


def _root_kernel(first, x, roots, idxv, rows, sem):
    cid = lax.axis_index("c")
    sid = lax.axis_index("s")

    @pl.when(jnp.logical_and(cid == 0, sid == 0))
    def _():
        pltpu.sync_copy(first, idxv)
        pltpu.async_copy(x.at[idxv], rows, sem).wait()
        pltpu.sync_copy(rows, roots)


def _sc_root_gather(first, x):
    k = pl.kernel(
        _root_kernel,
        out_type=jax.ShapeDtypeStruct((G, F), jnp.float32),
        mesh=_sc_mesh(),
        scratch_types=[
            pltpu.VMEM((G,), jnp.int32),
            pltpu.VMEM((G, F), jnp.float32),
            pltpu.SemaphoreType.DMA,
        ],
    )
    return k(first, x)


def _agg_kernel(p, e0, e1, sp, dp, zrows,
                out, acc, rows_a, rows_b,
                sidx_0, sidx_1, sidx_2, sidx_3,
                didx_0, didx_1, didx_2, didx_3,
                gsem_a, gsem_b, ssem_a, ssem_b,
                isem_0, isem_1, isem_2, isem_3):
    cid = lax.axis_index("c")
    sid = lax.axis_index("s")
    pltpu.sync_copy(zrows, acc.at[pl.ds(sid * 640, 640)])
    plsc.subcore_barrier()

    for d in (0, 1):
        s_in = e0 if d == 0 else e1
        d_in = e1 if d == 0 else e0

        @pl.when(cid == d)
        def _():
            pd = p.at[d]
            rows = (rows_a, rows_b)
            gsem = (gsem_a, gsem_b)
            ssem = (ssem_a, ssem_b)
            sidx = (sidx_0, sidx_1, sidx_2, sidx_3)
            didx = (didx_0, didx_1, didx_2, didx_3)
            isem = (isem_0, isem_1, isem_2, isem_3)

            def fire_idx(j, ib):
                off = (sid * NCH + j) * ECH

                @pl.when(off < E)
                def _():
                    pltpu.async_copy(s_in.at[pl.ds(off, ECH)],
                                     sidx[ib], isem[ib])
                    pltpu.async_copy(d_in.at[pl.ds(off, ECH)],
                                     didx[ib], isem[ib])

                @pl.when(off >= E)
                def _():
                    poff = off - E
                    pltpu.async_copy(sp.at[pl.ds(poff, ECH)],
                                     sidx[ib], isem[ib])
                    pltpu.async_copy(dp.at[pl.ds(poff, ECH)],
                                     didx[ib], isem[ib])

            def wait_idx(ib):
                pltpu.make_async_copy(s_in.at[pl.ds(0, ECH)], sidx[ib],
                                      isem[ib]).wait()
                pltpu.make_async_copy(d_in.at[pl.ds(0, ECH)], didx[ib],
                                      isem[ib]).wait()

            def fire_gather(rb, ib):
                pltpu.async_copy(pd.at[sidx[ib]], rows[rb], gsem[rb])

            def wait_gather(rb, ib):
                pltpu.make_async_copy(pd.at[sidx[ib]], rows[rb],
                                      gsem[rb]).wait()

            def scatter(rb, ib):
                pltpu.sync_copy(rows[rb], acc.at[didx[ib]], add=True)

            fire_idx(0, 0)
            fire_idx(1, 1)
            wait_idx(0)
            fire_gather(0, 0)
            fire_idx(2, 2)

            @pl.loop(0, (NCH - 4) // 4)
            def _(j):
                for k in range(4):
                    c = 4 * j + k + 1
                    rb = 1 - k % 2
                    ib = (k + 1) % 4
                    wait_idx(ib)
                    fire_gather(rb, ib)
                    fire_idx(c + 2, (k + 3) % 4)
                    wait_gather(1 - rb, k % 4)
                    scatter(1 - rb, k % 4)

            wait_idx(1)
            fire_gather(1, 1)
            fire_idx(159, 3)
            wait_gather(0, 0)
            scatter(0, 0)
            wait_idx(2)
            fire_gather(0, 2)
            wait_gather(1, 1)
            scatter(1, 1)
            wait_idx(3)
            fire_gather(1, 3)
            wait_gather(0, 2)
            scatter(0, 2)
            wait_gather(1, 3)
            scatter(1, 3)

    plsc.subcore_barrier()

    for d in (0, 1):
        @pl.when(cid == d)
        def _():
            _writeout_split(
                lambda r0, n: pltpu.sync_copy(acc.at[pl.ds(r0, n)],
                                              out.at[d].at[pl.ds(r0, n)]))


def _sc_edge_agg(p, e0, e1, sp, dp, zrows):
    k = pl.kernel(
        _agg_kernel,
        out_type=jax.ShapeDtypeStruct((2, N, F), jnp.float32),
        mesh=_sc_mesh(),
        scratch_types=(
            [pltpu.VMEM_SHARED((NPAD, F), jnp.float32)]
            + [pltpu.VMEM((ECH, F), jnp.float32)] * 2
            + [pltpu.VMEM((ECH,), jnp.int32)] * 8
            + [pltpu.SemaphoreType.DMA] * 8
        ),
    )
    return k(p, e0, e1, sp, dp, zrows)


EB = 3200
EBLK = E // EB
DD = 100


def _hist_kernel(ei, dh):
    i = pl.program_id(0)

    @pl.when(i == 0)
    def _():
        dh[...] = jnp.zeros_like(dh)

    ia = lax.broadcasted_iota(jnp.int32, (DD, EB), 0)
    for d in (0, 1):
        dst = ei[1 - d:2 - d, :]
        a = dst // DD
        b = dst - a * DD
        oha = (a == ia).astype(jnp.bfloat16)
        ohb = (b == ia).astype(jnp.bfloat16)
        dh[d] += lax.dot_general(ohb, oha, (((1,), (1,)), ((), ())),
                                 preferred_element_type=jnp.float32)


def _tc_deghist(ei):
    return pl.pallas_call(
        _hist_kernel,
        grid=(EBLK,),
        in_specs=[pl.BlockSpec((2, EB), lambda i: (0, i))],
        out_specs=pl.BlockSpec((2, DD, DD), lambda i: (0, 0, 0)),
        out_shape=jax.ShapeDtypeStruct((2, DD, DD), jnp.float32),
    )(ei)


def _prep_kernel(ei, e0lin, e1lin):
    e0lin[...] = ei[0:1, :].reshape(E)
    e1lin[...] = ei[1:2, :].reshape(E)


def _tc_edgeprep(ei):
    return pl.pallas_call(
        _prep_kernel,
        out_shape=[jax.ShapeDtypeStruct((E,), jnp.int32)] * 2,
    )(ei)


def _extract_deg(dmat, i):
    lane = lax.broadcasted_iota(jnp.int32, (DD, DD), 1)
    cols = []
    for k in range(NB // DD):
        sel = (lane == (NB // DD) * i + k).astype(jnp.float32)
        cols.append(jnp.sum(dmat * sel, axis=1, keepdims=True))
    return jnp.concatenate(cols, axis=0)


def _hscale_kernel(x, wcat, dh, bc, p, dis, first, cnt, cacc):
    i = pl.program_id(0)
    hb = jnp.dot(x[...], wcat[...], preferred_element_type=jnp.float32)
    for d in (0, 1):
        deg = _extract_deg(dh[d], i)
        dv = jax.lax.rsqrt(deg + 1.0)
        p[d] = hb[:, d * F:(d + 1) * F] * dv
        dis[d] = dv

    @pl.when(i == 0)
    def _():
        cacc[...] = jnp.zeros_like(cacc)

    gidx = lax.broadcasted_iota(jnp.int32, (NB, G), 1)
    oh = (bc[...] == gidx).astype(jnp.float32)
    cacc[...] += jnp.sum(oh, axis=0, keepdims=True)

    @pl.when(i == NBLK - 1)
    def _():
        c = cacc[...]
        gj = lax.broadcasted_iota(jnp.int32, (G, G), 0)
        gg = lax.broadcasted_iota(jnp.int32, (G, G), 1)
        lt = (gj < gg).astype(jnp.float32)
        f = jnp.dot(c, lt, preferred_element_type=jnp.float32)
        first[...] = jnp.clip(f.astype(jnp.int32), 0, N - 1)
        cnt[...] = c


def _tc_hscale(x, wcat, dh, batch_c):
    return pl.pallas_call(
        _hscale_kernel,
        grid=(NBLK,),
        in_specs=[
            pl.BlockSpec((NB, F), lambda i: (i, 0)),
            pl.BlockSpec((F, 2 * F), lambda i: (0, 0)),
            pl.BlockSpec((2, DD, DD), lambda i: (0, 0, 0)),
            pl.BlockSpec((NB, 1), lambda i: (i, 0)),
        ],
        out_specs=[
            pl.BlockSpec((2, NB, F), lambda i: (0, i, 0)),
            pl.BlockSpec((2, NB, 1), lambda i: (0, i, 0)),
            pl.BlockSpec((1, G), lambda i: (0, 0)),
            pl.BlockSpec((1, G), lambda i: (0, 0)),
        ],
        out_shape=[
            jax.ShapeDtypeStruct((2, N, F), jnp.float32),
            jax.ShapeDtypeStruct((2, N, 1), jnp.float32),
            jax.ShapeDtypeStruct((1, G), jnp.int32),
            jax.ShapeDtypeStruct((1, G), jnp.float32),
        ],
        scratch_shapes=[pltpu.VMEM((1, G), jnp.float32)],
    )(x, wcat, dh, batch_c)


def _layer2_kernel(pp, agg, dis, bc, roots, wbot, wtop, b1, b2, p2, basev):
    q = jnp.dot(jax.nn.relu(roots[...]), wbot[...],
                preferred_element_type=jnp.float32)
    b = bc[...]
    gidx = lax.broadcasted_iota(jnp.int32, (NB, G), 1)
    oh = (b == gidx).astype(jnp.float32)
    for d in (0, 1):
        dd = dis[d]
        conv1 = dd * (agg[d] + pp[d]) + b1[d]
        t = jax.nn.relu(conv1)
        h2 = (jnp.dot(t, wtop[d], preferred_element_type=jnp.float32)
              + jnp.dot(oh, q[:, d * F:(d + 1) * F],
                        preferred_element_type=jnp.float32))
        p2d = dd * h2
        p2[d] = p2d
        basev[d] = dd * p2d + b2[d]


def _tc_layer2(p, agg, dis, batch_c, roots, wbot_cat, wtop, b1, b2):
    return pl.pallas_call(
        _layer2_kernel,
        grid=(NBLK,),
        in_specs=[
            pl.BlockSpec((2, NB, F), lambda i: (0, i, 0)),
            pl.BlockSpec((2, NB, F), lambda i: (0, i, 0)),
            pl.BlockSpec((2, NB, 1), lambda i: (0, i, 0)),
            pl.BlockSpec((NB, 1), lambda i: (i, 0)),
            pl.BlockSpec((G, F), lambda i: (0, 0)),
            pl.BlockSpec((F, 2 * F), lambda i: (0, 0)),
            pl.BlockSpec((2, F, F), lambda i: (0, 0, 0)),
            pl.BlockSpec((2, F), lambda i: (0, 0)),
            pl.BlockSpec((2, F), lambda i: (0, 0)),
        ],
        out_specs=[
            pl.BlockSpec((2, NB, F), lambda i: (0, i, 0)),
            pl.BlockSpec((2, NB, F), lambda i: (0, i, 0)),
        ],
        out_shape=[
            jax.ShapeDtypeStruct((2, N, F), jnp.float32),
            jax.ShapeDtypeStruct((2, N, F), jnp.float32),
        ],
    )(p, agg, dis, batch_c, roots, wbot_cat, wtop, b1, b2)


def _final_kernel(agg2, basev, dis, bc, cnt, lw, lb, out, acc):
    i = pl.program_id(0)

    @pl.when(i == 0)
    def _():
        acc[...] = jnp.zeros_like(acc)

    b = bc[...]
    gidx = lax.broadcasted_iota(jnp.int32, (NB, G), 1)
    oh = (b == gidx).astype(jnp.float32)
    for d in (0, 1):
        h = jax.nn.relu(dis[d] * agg2[d] + basev[d])
        contrib = lax.dot_general(oh, h, (((0,), (0,)), ((), ())),
                                  preferred_element_type=jnp.float32)
        acc[:, d * F:(d + 1) * F] += contrib

    @pl.when(i == NBLK - 1)
    def _():
        gr = lax.broadcasted_iota(jnp.int32, (G, G), 0)
        gc = lax.broadcasted_iota(jnp.int32, (G, G), 1)
        dm = (gr == gc).astype(jnp.float32) / jnp.maximum(cnt[...], 1.0)
        mean = jnp.dot(dm, acc[...], preferred_element_type=jnp.float32)
        logits = jnp.dot(mean, lw[...],
                         preferred_element_type=jnp.float32) + lb[...]
        m = jnp.max(logits, axis=-1, keepdims=True)
        z = logits - m
        out[...] = z - jnp.log(jnp.sum(jnp.exp(z), axis=-1, keepdims=True))


def _tc_final(agg2, basev, dis, batch_c, cnt, lw, lb):
    return pl.pallas_call(
        _final_kernel,
        grid=(NBLK,),
        in_specs=[
            pl.BlockSpec((2, NB, F), lambda i: (0, i, 0)),
            pl.BlockSpec((2, NB, F), lambda i: (0, i, 0)),
            pl.BlockSpec((2, NB, 1), lambda i: (0, i, 0)),
            pl.BlockSpec((NB, 1), lambda i: (i, 0)),
            pl.BlockSpec((1, G), lambda i: (0, 0)),
            pl.BlockSpec((2 * F, C), lambda i: (0, 0)),
            pl.BlockSpec((1, C), lambda i: (0, 0)),
        ],
        out_specs=pl.BlockSpec((G, C), lambda i: (0, 0)),
        out_shape=jax.ShapeDtypeStruct((G, C), jnp.float32),
        scratch_shapes=[pltpu.VMEM((G, 2 * F), jnp.float32)],
    )(agg2, basev, dis, batch_c, cnt, lw, lb)


def kernel(x, edge_index, batch, bu1_W, bu1_b, td1_W, td1_b, root_W, root_b,
           bu2_W, bu2_b, td2_W, td2_b, lin_W, lin_b):
    ei = edge_index.astype(jnp.int32)
    batch_c = batch.astype(jnp.int32).reshape(N, 1)
    zrows = jnp.zeros((640, F), jnp.float32)

    wcat = jnp.concatenate([bu1_W, td1_W], axis=1)
    wtop = jnp.stack([bu2_W[:F], td2_W[:F]])
    wbot_cat = jnp.concatenate([bu2_W[F:], td2_W[F:]], axis=1)
    b1 = jnp.stack([bu1_b, td1_b])
    b2 = jnp.stack([bu2_b, td2_b])

    dh = _tc_deghist(ei)
    e0lin, e1lin = _tc_edgeprep(ei)
    p, dis, first, cnt = _tc_hscale(x, wcat, dh, batch_c)

    roots = _sc_root_gather(first.reshape(G), x)

    it = jnp.arange(EPAD - E, dtype=jnp.int32)
    sp = it % 128
    dp = N + it % (NPAD - N)

    agg = _sc_edge_agg(p, e0lin, e1lin, sp, dp, zrows)
    p2, basev = _tc_layer2(p, agg, dis, batch_c, roots, wbot_cat, wtop, b1, b2)
    agg2 = _sc_edge_agg(p2, e0lin, e1lin, sp, dp, zrows)
    return _tc_final(agg2, basev, dis, batch_c, cnt,
                     lin_W, lin_b.reshape(1, C))


# speedup vs baseline: 1.0696x; 1.0033x over previous
import jax
import jax.numpy as jnp
from jax import lax
from jax.experimental import pallas as pl
from jax.experimental.pallas import tpu as pltpu
from jax.experimental.pallas import tpu_sc as plsc

N = 10000
E = 320000
F = 128
G = 64
C = 4
NB = 400
NBLK = N // NB
NSC = 2
NTEC = 16
ECH = 128
NCH = 160
EPAD = NTEC * NCH * ECH
NPAD = 10240


def _sc_mesh():
    return plsc.VectorSubcoreMesh(core_axis_name="c", subcore_axis_name="s",
                                  num_cores=NSC, num_subcores=NTEC)


def _writeout_split(copy_fn):
    sid = lax.axis_index("s")

    @pl.when(sid < NTEC - 1)
    def _():
        copy_fn(sid * 640, 640)

    @pl.when(sid == NTEC - 1)
    def _():
        copy_fn((NTEC - 1) * 640, N - (NTEC - 1) * 640)


def _root_kernel(first, x, roots, idxv, rows, sem):
    cid = lax.axis_index("c")
    sid = lax.axis_index("s")

    @pl.when(jnp.logical_and(cid == 0, sid == 0))
    def _():
        pltpu.sync_copy(first, idxv)
        pltpu.async_copy(x.at[idxv], rows, sem).wait()
        pltpu.sync_copy(rows, roots)


def _sc_root_gather(first, x):
    k = pl.kernel(
        _root_kernel,
        out_type=jax.ShapeDtypeStruct((G, F), jnp.float32),
        mesh=_sc_mesh(),
        scratch_types=[
            pltpu.VMEM((G,), jnp.int32),
            pltpu.VMEM((G, F), jnp.float32),
            pltpu.SemaphoreType.DMA,
        ],
    )
    return k(first, x)


def _agg_kernel(p, e0, e1, sp, dp, zrows,
                out, acc, rows_a, rows_b,
                sidx_0, sidx_1, sidx_2, sidx_3,
                didx_0, didx_1, didx_2, didx_3,
                gsem_a, gsem_b,
                isem_0, isem_1, isem_2, isem_3):
    cid = lax.axis_index("c")
    sid = lax.axis_index("s")
    pltpu.sync_copy(zrows, acc.at[pl.ds(sid * 640, 640)])
    plsc.subcore_barrier()

    for d in (0, 1):
        s_in = e0 if d == 0 else e1
        d_in = e1 if d == 0 else e0

        @pl.when(cid == d)
        def _():
            pd = p.at[d]
            rows = (rows_a, rows_b)
            gsem = (gsem_a, gsem_b)
            sidx = (sidx_0, sidx_1, sidx_2, sidx_3)
            didx = (didx_0, didx_1, didx_2, didx_3)
            isem = (isem_0, isem_1, isem_2, isem_3)

            def fire_idx(j, ib):
                off = (sid * NCH + j) * ECH

                @pl.when(off < E)
                def _():
                    pltpu.async_copy(s_in.at[pl.ds(off, ECH)],
                                     sidx[ib], isem[ib])
                    pltpu.async_copy(d_in.at[pl.ds(off, ECH)],
                                     didx[ib], isem[ib])

                @pl.when(off >= E)
                def _():
                    poff = off - E
                    pltpu.async_copy(sp.at[pl.ds(poff, ECH)],
                                     sidx[ib], isem[ib])
                    pltpu.async_copy(dp.at[pl.ds(poff, ECH)],
                                     didx[ib], isem[ib])

            def wait_idx(ib):
                pltpu.make_async_copy(s_in.at[pl.ds(0, ECH)], sidx[ib],
                                      isem[ib]).wait()
                pltpu.make_async_copy(d_in.at[pl.ds(0, ECH)], didx[ib],
                                      isem[ib]).wait()

            def fire_gather(rb, ib):
                pltpu.async_copy(pd.at[sidx[ib]], rows[rb], gsem[rb])

            def wait_gather(rb, ib):
                pltpu.make_async_copy(pd.at[sidx[ib]], rows[rb],
                                      gsem[rb]).wait()

            def scatter(rb, ib):
                pltpu.sync_copy(rows[rb], acc.at[didx[ib]], add=True)

            fire_idx(0, 0)
            fire_idx(1, 1)
            wait_idx(0)
            fire_gather(0, 0)
            fire_idx(2, 2)

            @pl.loop(0, (NCH - 4) // 4)
            def _(j):
                for k in range(4):
                    c = 4 * j + k + 1
                    rb = 1 - k % 2
                    ib = (k + 1) % 4
                    wait_idx(ib)
                    fire_gather(rb, ib)
                    fire_idx(c + 2, (k + 3) % 4)
                    wait_gather(1 - rb, k % 4)
                    scatter(1 - rb, k % 4)

            wait_idx(1)
            fire_gather(1, 1)
            fire_idx(159, 3)
            wait_gather(0, 0)
            scatter(0, 0)
            wait_idx(2)
            fire_gather(0, 2)
            wait_gather(1, 1)
            scatter(1, 1)
            wait_idx(3)
            fire_gather(1, 3)
            wait_gather(0, 2)
            scatter(0, 2)
            wait_gather(1, 3)
            scatter(1, 3)

    plsc.subcore_barrier()

    for d in (0, 1):
        @pl.when(cid == d)
        def _():
            _writeout_split(
                lambda r0, n: pltpu.sync_copy(acc.at[pl.ds(r0, n)],
                                              out.at[d].at[pl.ds(r0, n)]))


def _sc_edge_agg(p, e0, e1, sp, dp, zrows):
    k = pl.kernel(
        _agg_kernel,
        out_type=jax.ShapeDtypeStruct((2, N, F), jnp.float32),
        mesh=_sc_mesh(),
        scratch_types=(
            [pltpu.VMEM_SHARED((NPAD, F), jnp.float32)]
            + [pltpu.VMEM((ECH, F), jnp.float32)] * 2
            + [pltpu.VMEM((ECH,), jnp.int32)] * 8
            + [pltpu.SemaphoreType.DMA] * 6
        ),
    )
    return k(p, e0, e1, sp, dp, zrows)


EB = 3200
EBLK = E // EB
DD = 100


def _hist_kernel(ei, dh):
    i = pl.program_id(0)

    @pl.when(i == 0)
    def _():
        dh[...] = jnp.zeros_like(dh)

    ia = lax.broadcasted_iota(jnp.int32, (DD, EB), 0)
    for d in (0, 1):
        dst = ei[1 - d:2 - d, :]
        a = dst // DD
        b = dst - a * DD
        oha = (a == ia).astype(jnp.bfloat16)
        ohb = (b == ia).astype(jnp.bfloat16)
        dh[d] += lax.dot_general(ohb, oha, (((1,), (1,)), ((), ())),
                                 preferred_element_type=jnp.float32)


def _tc_deghist(ei):
    return pl.pallas_call(
        _hist_kernel,
        grid=(EBLK,),
        in_specs=[pl.BlockSpec((2, EB), lambda i: (0, i))],
        out_specs=pl.BlockSpec((2, DD, DD), lambda i: (0, 0, 0)),
        out_shape=jax.ShapeDtypeStruct((2, DD, DD), jnp.float32),
    )(ei)


def _prep_kernel(ei, e0lin, e1lin):
    e0lin[...] = ei[0:1, :].reshape(E)
    e1lin[...] = ei[1:2, :].reshape(E)


def _tc_edgeprep(ei):
    return pl.pallas_call(
        _prep_kernel,
        out_shape=[jax.ShapeDtypeStruct((E,), jnp.int32)] * 2,
    )(ei)


def _extract_deg(dmat, i):
    lane = lax.broadcasted_iota(jnp.int32, (DD, DD), 1)
    cols = []
    for k in range(NB // DD):
        sel = (lane == (NB // DD) * i + k).astype(jnp.float32)
        cols.append(jnp.sum(dmat * sel, axis=1, keepdims=True))
    return jnp.concatenate(cols, axis=0)


def _hscale_kernel(x, wcat, dh, bc, p, dis, first, cnt, cacc):
    i = pl.program_id(0)
    hb = jnp.dot(x[...], wcat[...], preferred_element_type=jnp.float32)
    for d in (0, 1):
        deg = _extract_deg(dh[d], i)
        dv = jax.lax.rsqrt(deg + 1.0)
        p[d] = hb[:, d * F:(d + 1) * F] * dv
        dis[d] = dv

    @pl.when(i == 0)
    def _():
        cacc[...] = jnp.zeros_like(cacc)

    gidx = lax.broadcasted_iota(jnp.int32, (NB, G), 1)
    oh = (bc[...] == gidx).astype(jnp.float32)
    cacc[...] += jnp.sum(oh, axis=0, keepdims=True)

    @pl.when(i == NBLK - 1)
    def _():
        c = cacc[...]
        gj = lax.broadcasted_iota(jnp.int32, (G, G), 0)
        gg = lax.broadcasted_iota(jnp.int32, (G, G), 1)
        lt = (gj < gg).astype(jnp.float32)
        f = jnp.dot(c, lt, preferred_element_type=jnp.float32)
        first[...] = jnp.clip(f.astype(jnp.int32), 0, N - 1)
        cnt[...] = c


def _tc_hscale(x, wcat, dh, batch_c):
    return pl.pallas_call(
        _hscale_kernel,
        grid=(NBLK,),
        in_specs=[
            pl.BlockSpec((NB, F), lambda i: (i, 0)),
            pl.BlockSpec((F, 2 * F), lambda i: (0, 0)),
            pl.BlockSpec((2, DD, DD), lambda i: (0, 0, 0)),
            pl.BlockSpec((NB, 1), lambda i: (i, 0)),
        ],
        out_specs=[
            pl.BlockSpec((2, NB, F), lambda i: (0, i, 0)),
            pl.BlockSpec((2, NB, 1), lambda i: (0, i, 0)),
            pl.BlockSpec((1, G), lambda i: (0, 0)),
            pl.BlockSpec((1, G), lambda i: (0, 0)),
        ],
        out_shape=[
            jax.ShapeDtypeStruct((2, N, F), jnp.float32),
            jax.ShapeDtypeStruct((2, N, 1), jnp.float32),
            jax.ShapeDtypeStruct((1, G), jnp.int32),
            jax.ShapeDtypeStruct((1, G), jnp.float32),
        ],
        scratch_shapes=[pltpu.VMEM((1, G), jnp.float32)],
    )(x, wcat, dh, batch_c)


def _layer2_kernel(pp, agg, dis, bc, roots, wbot, wtop, b1, b2, p2, basev):
    q = jnp.dot(jax.nn.relu(roots[...]), wbot[...],
                preferred_element_type=jnp.float32)
    b = bc[...]
    gidx = lax.broadcasted_iota(jnp.int32, (NB, G), 1)
    oh = (b == gidx).astype(jnp.float32)
    for d in (0, 1):
        dd = dis[d]
        conv1 = dd * (agg[d] + pp[d]) + b1[d]
        t = jax.nn.relu(conv1)
        h2 = (jnp.dot(t, wtop[d], preferred_element_type=jnp.float32)
              + jnp.dot(oh, q[:, d * F:(d + 1) * F],
                        preferred_element_type=jnp.float32))
        p2d = dd * h2
        p2[d] = p2d
        basev[d] = dd * p2d + b2[d]


def _tc_layer2(p, agg, dis, batch_c, roots, wbot_cat, wtop, b1, b2):
    return pl.pallas_call(
        _layer2_kernel,
        grid=(NBLK,),
        in_specs=[
            pl.BlockSpec((2, NB, F), lambda i: (0, i, 0)),
            pl.BlockSpec((2, NB, F), lambda i: (0, i, 0)),
            pl.BlockSpec((2, NB, 1), lambda i: (0, i, 0)),
            pl.BlockSpec((NB, 1), lambda i: (i, 0)),
            pl.BlockSpec((G, F), lambda i: (0, 0)),
            pl.BlockSpec((F, 2 * F), lambda i: (0, 0)),
            pl.BlockSpec((2, F, F), lambda i: (0, 0, 0)),
            pl.BlockSpec((2, F), lambda i: (0, 0)),
            pl.BlockSpec((2, F), lambda i: (0, 0)),
        ],
        out_specs=[
            pl.BlockSpec((2, NB, F), lambda i: (0, i, 0)),
            pl.BlockSpec((2, NB, F), lambda i: (0, i, 0)),
        ],
        out_shape=[
            jax.ShapeDtypeStruct((2, N, F), jnp.float32),
            jax.ShapeDtypeStruct((2, N, F), jnp.float32),
        ],
    )(p, agg, dis, batch_c, roots, wbot_cat, wtop, b1, b2)


def _final_kernel(agg2, basev, dis, bc, cnt, lw, lb, out, acc):
    i = pl.program_id(0)

    @pl.when(i == 0)
    def _():
        acc[...] = jnp.zeros_like(acc)

    b = bc[...]
    gidx = lax.broadcasted_iota(jnp.int32, (NB, G), 1)
    oh = (b == gidx).astype(jnp.float32)
    for d in (0, 1):
        h = jax.nn.relu(dis[d] * agg2[d] + basev[d])
        contrib = lax.dot_general(oh, h, (((0,), (0,)), ((), ())),
                                  preferred_element_type=jnp.float32)
        acc[:, d * F:(d + 1) * F] += contrib

    @pl.when(i == NBLK - 1)
    def _():
        gr = lax.broadcasted_iota(jnp.int32, (G, G), 0)
        gc = lax.broadcasted_iota(jnp.int32, (G, G), 1)
        dm = (gr == gc).astype(jnp.float32) / jnp.maximum(cnt[...], 1.0)
        mean = jnp.dot(dm, acc[...], preferred_element_type=jnp.float32)
        logits = jnp.dot(mean, lw[...],
                         preferred_element_type=jnp.float32) + lb[...]
        m = jnp.max(logits, axis=-1, keepdims=True)
        z = logits - m
        out[...] = z - jnp.log(jnp.sum(jnp.exp(z), axis=-1, keepdims=True))


def _tc_final(agg2, basev, dis, batch_c, cnt, lw, lb):
    return pl.pallas_call(
        _final_kernel,
        grid=(NBLK,),
        in_specs=[
            pl.BlockSpec((2, NB, F), lambda i: (0, i, 0)),
            pl.BlockSpec((2, NB, F), lambda i: (0, i, 0)),
            pl.BlockSpec((2, NB, 1), lambda i: (0, i, 0)),
            pl.BlockSpec((NB, 1), lambda i: (i, 0)),
            pl.BlockSpec((1, G), lambda i: (0, 0)),
            pl.BlockSpec((2 * F, C), lambda i: (0, 0)),
            pl.BlockSpec((1, C), lambda i: (0, 0)),
        ],
        out_specs=pl.BlockSpec((G, C), lambda i: (0, 0)),
        out_shape=jax.ShapeDtypeStruct((G, C), jnp.float32),
        scratch_shapes=[pltpu.VMEM((G, 2 * F), jnp.float32)],
    )(agg2, basev, dis, batch_c, cnt, lw, lb)


def kernel(x, edge_index, batch, bu1_W, bu1_b, td1_W, td1_b, root_W, root_b,
           bu2_W, bu2_b, td2_W, td2_b, lin_W, lin_b):
    ei = edge_index.astype(jnp.int32)
    batch_c = batch.astype(jnp.int32).reshape(N, 1)
    zrows = jnp.zeros((640, F), jnp.float32)

    wcat = jnp.concatenate([bu1_W, td1_W], axis=1)
    wtop = jnp.stack([bu2_W[:F], td2_W[:F]])
    wbot_cat = jnp.concatenate([bu2_W[F:], td2_W[F:]], axis=1)
    b1 = jnp.stack([bu1_b, td1_b])
    b2 = jnp.stack([bu2_b, td2_b])

    dh = _tc_deghist(ei)
    e0lin, e1lin = _tc_edgeprep(ei)
    p, dis, first, cnt = _tc_hscale(x, wcat, dh, batch_c)

    roots = _sc_root_gather(first.reshape(G), x)

    it = jnp.arange(EPAD - E, dtype=jnp.int32)
    sp = it % 128
    dp = N + it % (NPAD - N)

    agg = _sc_edge_agg(p, e0lin, e1lin, sp, dp, zrows)
    p2, basev = _tc_layer2(p, agg, dis, batch_c, roots, wbot_cat, wtop, b1, b2)
    agg2 = _sc_edge_agg(p2, e0lin, e1lin, sp, dp, zrows)
    return _tc_final(agg2, basev, dis, batch_c, cnt,
                     lin_W, lin_b.reshape(1, C))
